# Initial kernel scaffold; baseline (speedup 1.0000x reference)
import jax
import jax.numpy as jnp
from jax.experimental import pallas as pl


def kernel(x, W):
    raise NotImplementedError("write your pallas kernel here")


# speedup vs baseline: 2.8176x; 2.8176x over previous
import functools

import jax
import jax.numpy as jnp
from jax.experimental import pallas as pl
from jax.experimental.pallas import tpu as pltpu

_SPLITS = 4
_SYMBOLS = 1024
_SUBDIM = 64


def _pq_body(x_ref, w_ref, out_ref, err_ref, *, n_tokens, block_tokens):
    i = pl.program_id(0)
    err_local = jnp.float32(0.0)
    for s in range(_SPLITS):
        xi = x_ref[:, s * _SUBDIM:(s + 1) * _SUBDIM]
        w = w_ref[s]
        cbsq = jnp.sum(w * w, axis=1)
        xsq = jnp.sum(xi * xi, axis=1, keepdims=True)
        prod = jax.lax.dot_general(
            xi, w, (((1,), (1,)), ((), ())),
            preferred_element_type=jnp.float32)
        sc = (xsq + cbsq[None, :]) - 2.0 * prod
        m = jnp.min(sc, axis=1, keepdims=True)
        iota = jax.lax.broadcasted_iota(jnp.int32, sc.shape, 1)
        idx = jnp.min(jnp.where(sc == m, iota, _SYMBOLS), axis=1,
                      keepdims=True)
        onehot = (iota == idx).astype(jnp.float32)
        sym = jax.lax.dot_general(
            onehot, w, (((1,), (0,)), ((), ())),
            preferred_element_type=jnp.float32)
        out_ref[:, s * _SUBDIM:(s + 1) * _SUBDIM] = sym
        err_local = err_local + jnp.sum(m)

    scale = jnp.float32(1.25 / (n_tokens * _SUBDIM))

    contrib = jnp.full((1, 1), err_local * scale, dtype=jnp.float32)

    @pl.when(i == 0)
    def _():
        err_ref[...] = contrib

    @pl.when(i > 0)
    def _():
        err_ref[...] = err_ref[...] + contrib


@jax.jit
def kernel(x, W):
    B, T, F = x.shape
    n_tokens = B * T
    block_tokens = 1152
    n_blocks = n_tokens // block_tokens
    x2 = x.reshape(n_tokens, F)

    quant, err = pl.pallas_call(
        functools.partial(_pq_body, n_tokens=n_tokens,
                          block_tokens=block_tokens),
        grid=(n_blocks,),
        in_specs=[
            pl.BlockSpec((block_tokens, F), lambda i: (i, 0)),
            pl.BlockSpec((_SPLITS, _SYMBOLS, _SUBDIM), lambda i: (0, 0, 0)),
        ],
        out_specs=[
            pl.BlockSpec((block_tokens, F), lambda i: (i, 0)),
            pl.BlockSpec((1, 1), lambda i: (0, 0)),
        ],
        out_shape=[
            jax.ShapeDtypeStruct((n_tokens, F), jnp.float32),
            jax.ShapeDtypeStruct((1, 1), jnp.float32),
        ],
    )(x2, W)

    return quant.reshape(B, T, F), err[0, 0]
